# baseline (speedup 1.0000x reference)
import functools

import jax
import jax.numpy as jnp
from jax import lax
from jax.experimental import pallas as pl
from jax.experimental.pallas import tpu as pltpu
from jax.experimental.pallas import tpu_sc as plsc

B = 16384
D = 64
V = 10000
NC = 2
NS = 16
FW = 4
NFW = D // FW
NH = 2
HB = B // NH
CHB = 2048
NCHB = HB // CHB
L = 16

_mesh = plsc.VectorSubcoreMesh(core_axis_name="c", subcore_axis_name="s")


@functools.partial(
    pl.kernel,
    mesh=_mesh,
    out_type=jax.ShapeDtypeStruct((D, B), jnp.float32),
    scratch_types=[
        pltpu.VMEM((HB,), jnp.int32),
        pltpu.VMEM((FW, V), jnp.float32),
        pltpu.VMEM((2, FW, CHB), jnp.float32),
        pltpu.SemaphoreType.DMA,
        pltpu.SemaphoreType.DMA((FW,)),
        pltpu.SemaphoreType.DMA((2,)),
    ],
    compiler_params=pltpu.CompilerParams(needs_layout_passes=False),
)
def _gather_fm(idxT_hbm, tableT_hbm, outT_hbm, idx_v, tab_v, out_v,
               isem, tsem, ssem):
    wid = lax.axis_index("s") * NC + lax.axis_index("c")
    f0 = (wid % NFW) * FW
    b0 = (wid // NFW) * HB
    c_idx = pltpu.async_copy(idxT_hbm.at[0, pl.ds(b0, HB)], idx_v, isem)
    c_tab = [
        pltpu.async_copy(
            tableT_hbm.at[pl.ds(f0 + f, 1), :],
            tab_v.at[pl.ds(f, 1)],
            tsem.at[f],
        )
        for f in range(FW)
    ]
    c_idx.wait()
    stores = [None, None]
    for ch in range(NCHB):
        buf = ch % 2
        if stores[buf] is not None:
            stores[buf].wait()
        for f in range(FW):
            if ch == 0:
                c_tab[f].wait()

            @plsc.parallel_loop(0, CHB, step=L, unroll=2)
            def _body(i, _ch=ch, _buf=buf, _f=f):
                iv = idx_v[pl.ds(_ch * CHB + i, L)]
                vals = plsc.load_gather(
                    tab_v, [jnp.full((L,), _f, jnp.int32), iv])
                out_v[_buf, _f, pl.ds(i, L)] = vals

        stores[buf] = pltpu.async_copy(
            out_v.at[buf],
            outT_hbm.at[pl.ds(f0, FW), pl.ds(b0 + ch * CHB, CHB)],
            ssem.at[buf],
        )
    for s in stores:
        if s is not None:
            s.wait()


def kernel(inputs, bias):
    outT = _gather_fm(inputs.T.astype(jnp.int32), bias.T)
    return outT.T


# speedup vs baseline: 1.0976x; 1.0976x over previous
import functools

import jax
import jax.numpy as jnp
from jax import lax
from jax.experimental import pallas as pl
from jax.experimental.pallas import tpu as pltpu
from jax.experimental.pallas import tpu_sc as plsc

B = 16384
D = 64
V = 10000
NC = 2
NS = 16
FW = 4
NFW = D // FW
NH = 2
HB = B // NH
CHB = 2048
NCHB = HB // CHB
L = 16

_mesh = plsc.VectorSubcoreMesh(core_axis_name="c", subcore_axis_name="s")


@functools.partial(
    pl.kernel,
    mesh=_mesh,
    out_type=jax.ShapeDtypeStruct((D, B), jnp.float32),
    scratch_types=[
        pltpu.VMEM((HB,), jnp.int32),
        pltpu.VMEM((FW, V), jnp.float32),
        pltpu.VMEM((2, FW, CHB), jnp.float32),
        pltpu.SemaphoreType.DMA,
        pltpu.SemaphoreType.DMA,
        pltpu.SemaphoreType.DMA((2,)),
    ],
    compiler_params=pltpu.CompilerParams(needs_layout_passes=False),
)
def _gather_fm(idxT_hbm, tableT_hbm, outT_hbm, idx_v, tab_v, out_v,
               isem, tsem, ssem):
    wid = lax.axis_index("s") * NC + lax.axis_index("c")
    f0 = (wid % NFW) * FW
    b0 = (wid // NFW) * HB
    c_idx = pltpu.async_copy(idxT_hbm.at[0, pl.ds(b0, HB)], idx_v, isem)
    c_tab = pltpu.async_copy(tableT_hbm.at[pl.ds(f0, FW), :], tab_v, tsem)
    c_idx.wait()
    c_tab.wait()
    stores = [None, None]
    for ch in range(NCHB):
        buf = ch % 2
        if stores[buf] is not None:
            stores[buf].wait()

        @plsc.parallel_loop(0, CHB, step=L)
        def _body(i, _ch=ch, _buf=buf):
            iv = idx_v[pl.ds(_ch * CHB + i, L)]
            for f in range(FW):
                vals = plsc.load_gather(
                    tab_v, [jnp.full((L,), f, jnp.int32), iv])
                out_v[_buf, f, pl.ds(i, L)] = vals

        stores[buf] = pltpu.async_copy(
            out_v.at[buf],
            outT_hbm.at[pl.ds(f0, FW), pl.ds(b0 + ch * CHB, CHB)],
            ssem.at[buf],
        )
    for s in stores:
        if s is not None:
            s.wait()


def kernel(inputs, bias):
    outT = _gather_fm(inputs.T.astype(jnp.int32), bias.T)
    return outT.T
